# Initial kernel scaffold; baseline (speedup 1.0000x reference)
import jax
import jax.numpy as jnp
from jax.experimental import pallas as pl


def kernel(phrase_feat, rel_feat, rel_conn_mat, target_id, W_rel, b_rel, W_sub, b_sub, W_obj, b_obj, W_phr, b_phr):
    raise NotImplementedError("write your pallas kernel here")


# speedup vs baseline: 23.3209x; 23.3209x over previous
import jax
import jax.numpy as jnp
from jax import lax
from jax.experimental import pallas as pl
from jax.experimental.pallas import tpu as pltpu

N = 256
D = 256
E = 4096
TE = 512
EPS = 1e-06


def _stage1(tid_ref, sub_ref, obj_ref, phrase_ref, rel_ref,
            W_rel_ref, b_rel_ref, W_sub_ref, b_sub_ref,
            W_obj_ref, b_obj_ref,
            u_ref, atte_ref, win_row_ref, win_col_ref):
    f32 = jnp.float32
    step = pl.program_id(0)
    t = tid_ref[0]
    sub_c = sub_ref[:]
    obj_c = obj_ref[:]
    phrase = phrase_ref[:]

    P_rs = jnp.dot(phrase, W_rel_ref[0:D, :], preferred_element_type=f32)
    P_ro = jnp.dot(phrase, W_rel_ref[D:2 * D, :], preferred_element_type=f32)
    P_ss = jnp.dot(phrase, W_sub_ref[0:D, :], preferred_element_type=f32)
    P_oo = jnp.dot(phrase, W_obj_ref[0:D, :], preferred_element_type=f32)

    colN = lax.broadcasted_iota(jnp.int32, (TE, N), 1)
    a_hot = sub_c == colN
    b_hot = obj_c == colN
    A = a_hot.astype(f32)
    B = b_hot.astype(f32)

    u = (jnp.dot(A, P_rs, preferred_element_type=f32)
         + jnp.dot(B, P_ro, preferred_element_type=f32)
         + jnp.dot(rel_ref[:], W_rel_ref[2 * D:3 * D, :],
                   preferred_element_type=f32)
         + b_rel_ref[:])
    u_ref[:] = u

    ts = (jnp.dot(A, P_ss, preferred_element_type=f32)
          + jnp.dot(u, W_sub_ref[D:2 * D, :], preferred_element_type=f32)
          + b_sub_ref[:])
    to = (jnp.dot(B, P_oo, preferred_element_type=f32)
          + jnp.dot(u, W_obj_ref[D:2 * D, :], preferred_element_type=f32)
          + b_obj_ref[:])
    atte_ref[:] = jnp.sum(ts * to, axis=1, keepdims=True) * (1.0 / 16.0)

    eplus = (lax.broadcasted_iota(jnp.int32, (TE, N), 0)
             + step * TE + 1)
    m_row = jnp.where((sub_c == t) & b_hot, eplus, 0)
    m_col = jnp.where((obj_c == t) & a_hot, eplus, 0)
    win_row_t = jnp.max(m_row, axis=0, keepdims=True)
    win_col_t = jnp.max(m_col, axis=0, keepdims=True)

    @pl.when(step == 0)
    def _():
        win_row_ref[:] = win_row_t
        win_col_ref[:] = win_col_t

    @pl.when(step > 0)
    def _():
        win_row_ref[:] = jnp.maximum(win_row_ref[:], win_row_t)
        win_col_ref[:] = jnp.maximum(win_col_ref[:], win_col_t)


def _stage2(tid_ref, u_ref, atte_ref, phrase_ref, W_phr_ref, b_phr_ref,
            win_row_ref, win_col_ref, out_phrase_ref):
    f32 = jnp.float32
    t = tid_ref[0]
    phrase = phrase_ref[:]
    u = u_ref[:]
    atte = atte_ref[:]
    win_row = win_row_ref[:]
    win_col = win_col_ref[:]

    rowE = lax.broadcasted_iota(jnp.int32, (E, N), 0)
    g_row = ((rowE == win_row - 1) & (win_row > 0)).astype(f32)
    g_col = ((rowE == win_col - 1) & (win_col > 0)).astype(f32)

    dn = (((0,), (0,)), ((), ()))
    rel_row = lax.dot_general(g_row, u, dn, preferred_element_type=f32)
    rel_col = lax.dot_general(g_col, u, dn, preferred_element_type=f32)
    atte_row = jnp.sum(g_row * atte, axis=0, keepdims=True)
    atte_col = jnp.sum(g_col * atte, axis=0, keepdims=True)

    mask_row = (win_row > 0).astype(f32)
    mask_col = (win_col > 0).astype(f32)

    def _masked_softmax_row(vec, mask):
        mv = vec * mask
        mx = jnp.max(mv, axis=1, keepdims=True)
        ex = jnp.exp(mv - mx) * mask
        return ex / (jnp.sum(ex, axis=1, keepdims=True) + EPS)

    a_s = _masked_softmax_row(atte_row, mask_row)
    a_o = _masked_softmax_row(atte_col, mask_col)

    ctx1 = jnp.dot(a_s + a_o, phrase, preferred_element_type=f32)
    ctx2 = (jnp.dot(a_s, rel_row, preferred_element_type=f32)
            + jnp.dot(a_o, rel_col, preferred_element_type=f32))
    delta = (jnp.dot(ctx1, W_phr_ref[0:D, :], preferred_element_type=f32)
             + jnp.dot(ctx2, W_phr_ref[D:2 * D, :], preferred_element_type=f32)
             + b_phr_ref[:])

    is_t = (lax.broadcasted_iota(jnp.int32, (N, 1), 0) == t).astype(f32)
    out_phrase_ref[:] = phrase + is_t * delta


def kernel(phrase_feat, rel_feat, rel_conn_mat, target_id, W_rel, b_rel,
           W_sub, b_sub, W_obj, b_obj, W_phr, b_phr):
    tid = jnp.reshape(jnp.asarray(target_id, jnp.int32), (1,))
    sub_c = jnp.reshape(rel_conn_mat[0].astype(jnp.int32), (E, 1))
    obj_c = jnp.reshape(rel_conn_mat[1].astype(jnp.int32), (E, 1))
    b_rel2 = jnp.reshape(b_rel, (1, D))
    b_sub2 = jnp.reshape(b_sub, (1, D))
    b_obj2 = jnp.reshape(b_obj, (1, D))
    b_phr2 = jnp.reshape(b_phr, (1, D))

    n_tiles = E // TE
    u, atte, win_row, win_col = pl.pallas_call(
        _stage1,
        grid=(n_tiles,),
        out_shape=(
            jax.ShapeDtypeStruct((E, D), jnp.float32),
            jax.ShapeDtypeStruct((E, 1), jnp.float32),
            jax.ShapeDtypeStruct((1, N), jnp.int32),
            jax.ShapeDtypeStruct((1, N), jnp.int32),
        ),
        in_specs=[
            pl.BlockSpec(memory_space=pltpu.SMEM),
            pl.BlockSpec((TE, 1), lambda i: (i, 0)),
            pl.BlockSpec((TE, 1), lambda i: (i, 0)),
            pl.BlockSpec((N, D), lambda i: (0, 0)),
            pl.BlockSpec((TE, D), lambda i: (i, 0)),
            pl.BlockSpec((3 * D, D), lambda i: (0, 0)),
            pl.BlockSpec((1, D), lambda i: (0, 0)),
            pl.BlockSpec((2 * D, D), lambda i: (0, 0)),
            pl.BlockSpec((1, D), lambda i: (0, 0)),
            pl.BlockSpec((2 * D, D), lambda i: (0, 0)),
            pl.BlockSpec((1, D), lambda i: (0, 0)),
        ],
        out_specs=(
            pl.BlockSpec((TE, D), lambda i: (i, 0)),
            pl.BlockSpec((TE, 1), lambda i: (i, 0)),
            pl.BlockSpec((1, N), lambda i: (0, 0)),
            pl.BlockSpec((1, N), lambda i: (0, 0)),
        ),
    )(tid, sub_c, obj_c, phrase_feat, rel_feat, W_rel, b_rel2,
      W_sub, b_sub2, W_obj, b_obj2)

    out_phrase = pl.pallas_call(
        _stage2,
        out_shape=jax.ShapeDtypeStruct((N, D), jnp.float32),
        in_specs=[pl.BlockSpec(memory_space=pltpu.SMEM)]
        + [pl.BlockSpec(memory_space=pltpu.VMEM)] * 7,
    )(tid, u, atte, phrase_feat, W_phr, b_phr2, win_row, win_col)

    return (out_phrase, u)


# speedup vs baseline: 26.5792x; 1.1397x over previous
import jax
import jax.numpy as jnp
from jax import lax
from jax.experimental import pallas as pl
from jax.experimental.pallas import tpu as pltpu

N = 256
D = 256
E = 4096
TE = 512
EPS = 1e-06
_NT = E // TE


def _sg_kernel(tid_ref, sub_ref, obj_ref, phrase_ref, rel_ref,
               W_rel_ref, b_rel_ref, W_sub_ref, b_sub_ref,
               W_obj_ref, b_obj_ref, W_phr_ref, b_phr_ref,
               u_ref, out_phrase_ref,
               u_s, atte_s, win_row_s, win_col_s):
    f32 = jnp.float32
    step = pl.program_id(0)
    t = tid_ref[0]
    sub_c = sub_ref[:]
    obj_c = obj_ref[:]
    phrase = phrase_ref[:]

    P_rs = jnp.dot(phrase, W_rel_ref[0:D, :], preferred_element_type=f32)
    P_ro = jnp.dot(phrase, W_rel_ref[D:2 * D, :], preferred_element_type=f32)
    P_ss = jnp.dot(phrase, W_sub_ref[0:D, :], preferred_element_type=f32)
    P_oo = jnp.dot(phrase, W_obj_ref[0:D, :], preferred_element_type=f32)

    colN = lax.broadcasted_iota(jnp.int32, (TE, N), 1)
    a_hot = sub_c == colN
    b_hot = obj_c == colN
    A = a_hot.astype(f32)
    B = b_hot.astype(f32)

    u = (jnp.dot(A, P_rs, preferred_element_type=f32)
         + jnp.dot(B, P_ro, preferred_element_type=f32)
         + jnp.dot(rel_ref[:], W_rel_ref[2 * D:3 * D, :],
                   preferred_element_type=f32)
         + b_rel_ref[:])
    u_ref[:] = u
    u_s[pl.ds(step * TE, TE), :] = u

    ts = (jnp.dot(A, P_ss, preferred_element_type=f32)
          + jnp.dot(u, W_sub_ref[D:2 * D, :], preferred_element_type=f32)
          + b_sub_ref[:])
    to = (jnp.dot(B, P_oo, preferred_element_type=f32)
          + jnp.dot(u, W_obj_ref[D:2 * D, :], preferred_element_type=f32)
          + b_obj_ref[:])
    atte_s[pl.ds(step * TE, TE), :] = (
        jnp.sum(ts * to, axis=1, keepdims=True) * (1.0 / 16.0))

    eplus = (lax.broadcasted_iota(jnp.int32, (TE, N), 0)
             + step * TE + 1)
    m_row = jnp.where((sub_c == t) & b_hot, eplus, 0)
    m_col = jnp.where((obj_c == t) & a_hot, eplus, 0)
    win_row_t = jnp.max(m_row, axis=0, keepdims=True)
    win_col_t = jnp.max(m_col, axis=0, keepdims=True)

    @pl.when(step == 0)
    def _():
        win_row_s[:] = win_row_t
        win_col_s[:] = win_col_t

    @pl.when(step > 0)
    def _():
        win_row_s[:] = jnp.maximum(win_row_s[:], win_row_t)
        win_col_s[:] = jnp.maximum(win_col_s[:], win_col_t)

    @pl.when(step == _NT - 1)
    def _():
        u_all = u_s[:]
        atte = atte_s[:]
        win_row = win_row_s[:]
        win_col = win_col_s[:]

        rowE = lax.broadcasted_iota(jnp.int32, (E, N), 0)
        g_row = ((rowE == win_row - 1) & (win_row > 0)).astype(f32)
        g_col = ((rowE == win_col - 1) & (win_col > 0)).astype(f32)

        dn = (((0,), (0,)), ((), ()))
        rel_row = lax.dot_general(g_row, u_all, dn,
                                  preferred_element_type=f32)
        rel_col = lax.dot_general(g_col, u_all, dn,
                                  preferred_element_type=f32)
        atte_row = jnp.sum(g_row * atte, axis=0, keepdims=True)
        atte_col = jnp.sum(g_col * atte, axis=0, keepdims=True)

        mask_row = (win_row > 0).astype(f32)
        mask_col = (win_col > 0).astype(f32)

        def _masked_softmax_row(vec, mask):
            mv = vec * mask
            mx = jnp.max(mv, axis=1, keepdims=True)
            ex = jnp.exp(mv - mx) * mask
            return ex / (jnp.sum(ex, axis=1, keepdims=True) + EPS)

        a_s = _masked_softmax_row(atte_row, mask_row)
        a_o = _masked_softmax_row(atte_col, mask_col)

        ctx1 = jnp.dot(a_s + a_o, phrase, preferred_element_type=f32)
        ctx2 = (jnp.dot(a_s, rel_row, preferred_element_type=f32)
                + jnp.dot(a_o, rel_col, preferred_element_type=f32))
        delta = (jnp.dot(ctx1, W_phr_ref[0:D, :], preferred_element_type=f32)
                 + jnp.dot(ctx2, W_phr_ref[D:2 * D, :],
                           preferred_element_type=f32)
                 + b_phr_ref[:])

        is_t = (lax.broadcasted_iota(jnp.int32, (N, 1), 0) == t).astype(f32)
        out_phrase_ref[:] = phrase + is_t * delta


def kernel(phrase_feat, rel_feat, rel_conn_mat, target_id, W_rel, b_rel,
           W_sub, b_sub, W_obj, b_obj, W_phr, b_phr):
    tid = jnp.reshape(jnp.asarray(target_id, jnp.int32), (1,))
    sub_c = jnp.reshape(rel_conn_mat[0].astype(jnp.int32), (E, 1))
    obj_c = jnp.reshape(rel_conn_mat[1].astype(jnp.int32), (E, 1))
    b_rel2 = jnp.reshape(b_rel, (1, D))
    b_sub2 = jnp.reshape(b_sub, (1, D))
    b_obj2 = jnp.reshape(b_obj, (1, D))
    b_phr2 = jnp.reshape(b_phr, (1, D))

    u, out_phrase = pl.pallas_call(
        _sg_kernel,
        grid=(_NT,),
        out_shape=(
            jax.ShapeDtypeStruct((E, D), jnp.float32),
            jax.ShapeDtypeStruct((N, D), jnp.float32),
        ),
        in_specs=[
            pl.BlockSpec(memory_space=pltpu.SMEM),
            pl.BlockSpec((TE, 1), lambda i: (i, 0)),
            pl.BlockSpec((TE, 1), lambda i: (i, 0)),
            pl.BlockSpec((N, D), lambda i: (0, 0)),
            pl.BlockSpec((TE, D), lambda i: (i, 0)),
            pl.BlockSpec((3 * D, D), lambda i: (0, 0)),
            pl.BlockSpec((1, D), lambda i: (0, 0)),
            pl.BlockSpec((2 * D, D), lambda i: (0, 0)),
            pl.BlockSpec((1, D), lambda i: (0, 0)),
            pl.BlockSpec((2 * D, D), lambda i: (0, 0)),
            pl.BlockSpec((1, D), lambda i: (0, 0)),
            pl.BlockSpec((2 * D, D), lambda i: (0, 0)),
            pl.BlockSpec((1, D), lambda i: (0, 0)),
        ],
        out_specs=(
            pl.BlockSpec((TE, D), lambda i: (i, 0)),
            pl.BlockSpec((N, D), lambda i: (0, 0)),
        ),
        scratch_shapes=[
            pltpu.VMEM((E, D), jnp.float32),
            pltpu.VMEM((E, 1), jnp.float32),
            pltpu.VMEM((1, N), jnp.int32),
            pltpu.VMEM((1, N), jnp.int32),
        ],
    )(tid, sub_c, obj_c, phrase_feat, rel_feat, W_rel, b_rel2,
      W_sub, b_sub2, W_obj, b_obj2, W_phr, b_phr2)

    return (out_phrase, u)


# speedup vs baseline: 37.1162x; 1.3964x over previous
import jax
import jax.numpy as jnp
from jax import lax
from jax.experimental import pallas as pl
from jax.experimental.pallas import tpu as pltpu

N = 256
D = 256
E = 4096
TE = 1024
EPS = 1e-06
_NT = E // TE


def _sg_kernel(tid_ref, sub_ref, obj_ref, phrase_ref, rel_ref,
               W_rel_ref, b_rel_ref, W_sub_ref, b_sub_ref,
               W_obj_ref, b_obj_ref, W_phr_ref, b_phr_ref,
               u_ref, out_phrase_ref,
               P_rs_s, P_ro_s, P_ss_s, P_oo_s,
               u_s, atte_s, win_row_s, win_col_s):
    f32 = jnp.float32
    step = pl.program_id(0)
    t = tid_ref[0]
    sub_r = sub_ref[0]
    obj_r = obj_ref[0]
    phrase = phrase_ref[:]

    @pl.when(step == 0)
    def _():
        P_rs_s[:] = jnp.dot(phrase, W_rel_ref[0:D, :],
                            preferred_element_type=f32)
        P_ro_s[:] = jnp.dot(phrase, W_rel_ref[D:2 * D, :],
                            preferred_element_type=f32)
        P_ss_s[:] = jnp.dot(phrase, W_sub_ref[0:D, :],
                            preferred_element_type=f32)
        P_oo_s[:] = jnp.dot(phrase, W_obj_ref[0:D, :],
                            preferred_element_type=f32)

    rowN = lax.broadcasted_iota(jnp.int32, (N, TE), 0)
    a_hot = sub_r == rowN
    b_hot = obj_r == rowN
    AT = a_hot.astype(f32)
    BT = b_hot.astype(f32)
    dn0 = (((0,), (0,)), ((), ()))

    u = (lax.dot_general(AT, P_rs_s[:], dn0, preferred_element_type=f32)
         + lax.dot_general(BT, P_ro_s[:], dn0, preferred_element_type=f32)
         + jnp.dot(rel_ref[:], W_rel_ref[2 * D:3 * D, :],
                   preferred_element_type=f32)
         + b_rel_ref[:])
    u_ref[:] = u
    u_s[pl.ds(step * TE, TE), :] = u

    ts = (lax.dot_general(AT, P_ss_s[:], dn0, preferred_element_type=f32)
          + jnp.dot(u, W_sub_ref[D:2 * D, :], preferred_element_type=f32)
          + b_sub_ref[:])
    to = (lax.dot_general(BT, P_oo_s[:], dn0, preferred_element_type=f32)
          + jnp.dot(u, W_obj_ref[D:2 * D, :], preferred_element_type=f32)
          + b_obj_ref[:])
    atte_s[pl.ds(step * TE, TE), :] = (
        jnp.sum(ts * to, axis=1, keepdims=True) * (1.0 / 16.0))

    eplus = (lax.broadcasted_iota(jnp.int32, (N, TE), 1)
             + step * TE + 1)
    m_row = jnp.where(b_hot & (sub_r == t), eplus, 0)
    m_col = jnp.where(a_hot & (obj_r == t), eplus, 0)
    win_row_t = jnp.max(m_row, axis=1, keepdims=True)
    win_col_t = jnp.max(m_col, axis=1, keepdims=True)

    @pl.when(step == 0)
    def _():
        win_row_s[:] = win_row_t
        win_col_s[:] = win_col_t

    @pl.when(step > 0)
    def _():
        win_row_s[:] = jnp.maximum(win_row_s[:], win_row_t)
        win_col_s[:] = jnp.maximum(win_col_s[:], win_col_t)

    @pl.when(step == _NT - 1)
    def _():
        u_all = u_s[:]
        atte = atte_s[:]
        win_row = win_row_s[:]
        win_col = win_col_s[:]

        colE = lax.broadcasted_iota(jnp.int32, (N, E), 1)
        g_row = ((colE == win_row - 1) & (win_row > 0)).astype(f32)
        g_col = ((colE == win_col - 1) & (win_col > 0)).astype(f32)

        rel_row = jnp.dot(g_row, u_all, preferred_element_type=f32)
        rel_col = jnp.dot(g_col, u_all, preferred_element_type=f32)
        atte_row = jnp.dot(g_row, atte, preferred_element_type=f32)
        atte_col = jnp.dot(g_col, atte, preferred_element_type=f32)

        mask_row = (win_row > 0).astype(f32)
        mask_col = (win_col > 0).astype(f32)

        def _masked_softmax(vec, mask):
            mv = vec * mask
            mx = jnp.max(mv, axis=0, keepdims=True)
            ex = jnp.exp(mv - mx) * mask
            return ex / (jnp.sum(ex, axis=0, keepdims=True) + EPS)

        a_s = _masked_softmax(atte_row, mask_row)
        a_o = _masked_softmax(atte_col, mask_col)

        ctx1 = jnp.sum(phrase * (a_s + a_o), axis=0, keepdims=True)
        ctx2 = jnp.sum(rel_row * a_s + rel_col * a_o, axis=0, keepdims=True)
        delta = (jnp.dot(ctx1, W_phr_ref[0:D, :], preferred_element_type=f32)
                 + jnp.dot(ctx2, W_phr_ref[D:2 * D, :],
                           preferred_element_type=f32)
                 + b_phr_ref[:])

        is_t = (lax.broadcasted_iota(jnp.int32, (N, 1), 0) == t).astype(f32)
        out_phrase_ref[:] = phrase + is_t * delta


def kernel(phrase_feat, rel_feat, rel_conn_mat, target_id, W_rel, b_rel,
           W_sub, b_sub, W_obj, b_obj, W_phr, b_phr):
    tid = jnp.reshape(jnp.asarray(target_id, jnp.int32), (1,))
    conn = jnp.reshape(rel_conn_mat.astype(jnp.int32), (2, 1, E))
    b_rel2 = jnp.reshape(b_rel, (1, D))
    b_sub2 = jnp.reshape(b_sub, (1, D))
    b_obj2 = jnp.reshape(b_obj, (1, D))
    b_phr2 = jnp.reshape(b_phr, (1, D))

    u, out_phrase = pl.pallas_call(
        _sg_kernel,
        grid=(_NT,),
        out_shape=(
            jax.ShapeDtypeStruct((E, D), jnp.float32),
            jax.ShapeDtypeStruct((N, D), jnp.float32),
        ),
        in_specs=[
            pl.BlockSpec(memory_space=pltpu.SMEM),
            pl.BlockSpec((1, 1, TE), lambda i: (0, 0, i)),
            pl.BlockSpec((1, 1, TE), lambda i: (1, 0, i)),
            pl.BlockSpec((N, D), lambda i: (0, 0)),
            pl.BlockSpec((TE, D), lambda i: (i, 0)),
            pl.BlockSpec((3 * D, D), lambda i: (0, 0)),
            pl.BlockSpec((1, D), lambda i: (0, 0)),
            pl.BlockSpec((2 * D, D), lambda i: (0, 0)),
            pl.BlockSpec((1, D), lambda i: (0, 0)),
            pl.BlockSpec((2 * D, D), lambda i: (0, 0)),
            pl.BlockSpec((1, D), lambda i: (0, 0)),
            pl.BlockSpec((2 * D, D), lambda i: (0, 0)),
            pl.BlockSpec((1, D), lambda i: (0, 0)),
        ],
        out_specs=(
            pl.BlockSpec((TE, D), lambda i: (i, 0)),
            pl.BlockSpec((N, D), lambda i: (0, 0)),
        ),
        scratch_shapes=[
            pltpu.VMEM((N, D), jnp.float32),
            pltpu.VMEM((N, D), jnp.float32),
            pltpu.VMEM((N, D), jnp.float32),
            pltpu.VMEM((N, D), jnp.float32),
            pltpu.VMEM((E, D), jnp.float32),
            pltpu.VMEM((E, 1), jnp.float32),
            pltpu.VMEM((N, 1), jnp.int32),
            pltpu.VMEM((N, 1), jnp.int32),
        ],
    )(tid, conn, conn, phrase_feat, rel_feat, W_rel, b_rel2,
      W_sub, b_sub2, W_obj, b_obj2, W_phr, b_phr2)

    return (out_phrase, u)


# speedup vs baseline: 38.1847x; 1.0288x over previous
import jax
import jax.numpy as jnp
from jax import lax
from jax.experimental import pallas as pl
from jax.experimental.pallas import tpu as pltpu

N = 256
D = 256
E = 4096
TE = 2048
EPS = 1e-06
_NT = E // TE


def _sg_kernel(tid_ref, sub_ref, obj_ref, phrase_ref, rel_ref,
               W_rel_ref, b_rel_ref, W_sub_ref, b_sub_ref,
               W_obj_ref, b_obj_ref, W_phr_ref, b_phr_ref,
               u_ref, out_phrase_ref,
               P_rs_s, P_ro_s, P_ss_s, P_oo_s,
               u_s, atte_s, win_row_s, win_col_s):
    f32 = jnp.float32
    step = pl.program_id(0)
    t = tid_ref[0]
    sub_r = sub_ref[0]
    obj_r = obj_ref[0]
    phrase = phrase_ref[:]

    @pl.when(step == 0)
    def _():
        P_rs_s[:] = jnp.dot(phrase, W_rel_ref[0:D, :],
                            preferred_element_type=f32)
        P_ro_s[:] = jnp.dot(phrase, W_rel_ref[D:2 * D, :],
                            preferred_element_type=f32)
        P_ss_s[:] = jnp.dot(phrase, W_sub_ref[0:D, :],
                            preferred_element_type=f32)
        P_oo_s[:] = jnp.dot(phrase, W_obj_ref[0:D, :],
                            preferred_element_type=f32)

    rowN = lax.broadcasted_iota(jnp.int32, (N, TE), 0)
    a_hot = sub_r == rowN
    b_hot = obj_r == rowN
    AT = a_hot.astype(f32)
    BT = b_hot.astype(f32)
    dn0 = (((0,), (0,)), ((), ()))

    u = (lax.dot_general(AT, P_rs_s[:], dn0, preferred_element_type=f32)
         + lax.dot_general(BT, P_ro_s[:], dn0, preferred_element_type=f32)
         + jnp.dot(rel_ref[:], W_rel_ref[2 * D:3 * D, :],
                   preferred_element_type=f32)
         + b_rel_ref[:])
    u_ref[:] = u
    u_s[pl.ds(step * TE, TE), :] = u

    ts = (lax.dot_general(AT, P_ss_s[:], dn0, preferred_element_type=f32)
          + jnp.dot(u, W_sub_ref[D:2 * D, :], preferred_element_type=f32)
          + b_sub_ref[:])
    to = (lax.dot_general(BT, P_oo_s[:], dn0, preferred_element_type=f32)
          + jnp.dot(u, W_obj_ref[D:2 * D, :], preferred_element_type=f32)
          + b_obj_ref[:])
    atte_s[pl.ds(step * TE, TE), :] = (
        jnp.sum(ts * to, axis=1, keepdims=True) * (1.0 / 16.0))

    eplus = (lax.broadcasted_iota(jnp.int32, (1, TE), 1)
             + step * TE + 1)
    e_row = jnp.where(sub_r == t, eplus, 0)
    e_col = jnp.where(obj_r == t, eplus, 0)
    m_row = jnp.where(b_hot, e_row, 0)
    m_col = jnp.where(a_hot, e_col, 0)
    win_row_t = jnp.max(m_row, axis=1, keepdims=True)
    win_col_t = jnp.max(m_col, axis=1, keepdims=True)

    @pl.when(step == 0)
    def _():
        win_row_s[:] = win_row_t
        win_col_s[:] = win_col_t

    @pl.when(step > 0)
    def _():
        win_row_s[:] = jnp.maximum(win_row_s[:], win_row_t)
        win_col_s[:] = jnp.maximum(win_col_s[:], win_col_t)

    @pl.when(step == _NT - 1)
    def _():
        u_all = u_s[:]
        atte = atte_s[:]
        win_row = win_row_s[:]
        win_col = win_col_s[:]

        colE = lax.broadcasted_iota(jnp.int32, (N, E), 1)
        g_row = ((colE == win_row - 1) & (win_row > 0)).astype(f32)
        g_col = ((colE == win_col - 1) & (win_col > 0)).astype(f32)

        rel_row = jnp.dot(g_row, u_all, preferred_element_type=f32)
        rel_col = jnp.dot(g_col, u_all, preferred_element_type=f32)
        atte_row = jnp.dot(g_row, atte, preferred_element_type=f32)
        atte_col = jnp.dot(g_col, atte, preferred_element_type=f32)

        mask_row = (win_row > 0).astype(f32)
        mask_col = (win_col > 0).astype(f32)

        def _masked_softmax(vec, mask):
            mv = vec * mask
            mx = jnp.max(mv, axis=0, keepdims=True)
            ex = jnp.exp(mv - mx) * mask
            return ex / (jnp.sum(ex, axis=0, keepdims=True) + EPS)

        a_s = _masked_softmax(atte_row, mask_row)
        a_o = _masked_softmax(atte_col, mask_col)

        ctx1 = jnp.sum(phrase * (a_s + a_o), axis=0, keepdims=True)
        ctx2 = jnp.sum(rel_row * a_s + rel_col * a_o, axis=0, keepdims=True)
        delta = (jnp.dot(ctx1, W_phr_ref[0:D, :], preferred_element_type=f32)
                 + jnp.dot(ctx2, W_phr_ref[D:2 * D, :],
                           preferred_element_type=f32)
                 + b_phr_ref[:])

        is_t = (lax.broadcasted_iota(jnp.int32, (N, 1), 0) == t).astype(f32)
        out_phrase_ref[:] = phrase + is_t * delta


def kernel(phrase_feat, rel_feat, rel_conn_mat, target_id, W_rel, b_rel,
           W_sub, b_sub, W_obj, b_obj, W_phr, b_phr):
    tid = jnp.reshape(jnp.asarray(target_id, jnp.int32), (1,))
    conn = jnp.reshape(rel_conn_mat.astype(jnp.int32), (2, 1, E))
    b_rel2 = jnp.reshape(b_rel, (1, D))
    b_sub2 = jnp.reshape(b_sub, (1, D))
    b_obj2 = jnp.reshape(b_obj, (1, D))
    b_phr2 = jnp.reshape(b_phr, (1, D))

    u, out_phrase = pl.pallas_call(
        _sg_kernel,
        grid=(_NT,),
        out_shape=(
            jax.ShapeDtypeStruct((E, D), jnp.float32),
            jax.ShapeDtypeStruct((N, D), jnp.float32),
        ),
        in_specs=[
            pl.BlockSpec(memory_space=pltpu.SMEM),
            pl.BlockSpec((1, 1, TE), lambda i: (0, 0, i)),
            pl.BlockSpec((1, 1, TE), lambda i: (1, 0, i)),
            pl.BlockSpec((N, D), lambda i: (0, 0)),
            pl.BlockSpec((TE, D), lambda i: (i, 0)),
            pl.BlockSpec((3 * D, D), lambda i: (0, 0)),
            pl.BlockSpec((1, D), lambda i: (0, 0)),
            pl.BlockSpec((2 * D, D), lambda i: (0, 0)),
            pl.BlockSpec((1, D), lambda i: (0, 0)),
            pl.BlockSpec((2 * D, D), lambda i: (0, 0)),
            pl.BlockSpec((1, D), lambda i: (0, 0)),
            pl.BlockSpec((2 * D, D), lambda i: (0, 0)),
            pl.BlockSpec((1, D), lambda i: (0, 0)),
        ],
        out_specs=(
            pl.BlockSpec((TE, D), lambda i: (i, 0)),
            pl.BlockSpec((N, D), lambda i: (0, 0)),
        ),
        scratch_shapes=[
            pltpu.VMEM((N, D), jnp.float32),
            pltpu.VMEM((N, D), jnp.float32),
            pltpu.VMEM((N, D), jnp.float32),
            pltpu.VMEM((N, D), jnp.float32),
            pltpu.VMEM((E, D), jnp.float32),
            pltpu.VMEM((E, 1), jnp.float32),
            pltpu.VMEM((N, 1), jnp.int32),
            pltpu.VMEM((N, 1), jnp.int32),
        ],
    )(tid, conn, conn, phrase_feat, rel_feat, W_rel, b_rel2,
      W_sub, b_sub2, W_obj, b_obj2, W_phr, b_phr2)

    return (out_phrase, u)


# speedup vs baseline: 41.9166x; 1.0977x over previous
import jax
import jax.numpy as jnp
from jax import lax
from jax.experimental import pallas as pl
from jax.experimental.pallas import tpu as pltpu

N = 256
D = 256
E = 4096
TE = 2048
EPS = 1e-06
_NT = E // TE


def _sg_kernel(tid_ref, sub_ref, obj_ref, phrase_ref, rel_ref,
               W_rel_ref, b_rel_ref, W_sub_ref, b_sub_ref,
               W_obj_ref, b_obj_ref, W_phr_ref, b_phr_ref,
               u_ref, out_phrase_ref,
               P_a_s, P_b_s, W_so_s,
               u_s, atte_s, win_row_s, win_col_s):
    f32 = jnp.float32
    step = pl.program_id(0)
    t = tid_ref[0]
    sub_r = sub_ref[0]
    obj_r = obj_ref[0]
    phrase = phrase_ref[:]

    @pl.when(step == 0)
    def _():
        P_a_s[:, 0:D] = jnp.dot(phrase, W_rel_ref[0:D, :],
                                preferred_element_type=f32)
        P_a_s[:, D:2 * D] = jnp.dot(phrase, W_sub_ref[0:D, :],
                                    preferred_element_type=f32)
        P_b_s[:, 0:D] = jnp.dot(phrase, W_rel_ref[D:2 * D, :],
                                preferred_element_type=f32)
        P_b_s[:, D:2 * D] = jnp.dot(phrase, W_obj_ref[0:D, :],
                                    preferred_element_type=f32)
        W_so_s[:, 0:D] = W_sub_ref[D:2 * D, :]
        W_so_s[:, D:2 * D] = W_obj_ref[D:2 * D, :]

    rowN = lax.broadcasted_iota(jnp.int32, (N, TE), 0)
    a_hot = sub_r == rowN
    b_hot = obj_r == rowN
    AT = a_hot.astype(f32)
    BT = b_hot.astype(f32)
    dn0 = (((0,), (0,)), ((), ()))

    Ra = lax.dot_general(AT, P_a_s[:], dn0, preferred_element_type=f32)
    Rb = lax.dot_general(BT, P_b_s[:], dn0, preferred_element_type=f32)
    u = (Ra[:, 0:D] + Rb[:, 0:D]
         + jnp.dot(rel_ref[:], W_rel_ref[2 * D:3 * D, :],
                   preferred_element_type=f32)
         + b_rel_ref[:])
    u_ref[:] = u
    u_s[pl.ds(step * TE, TE), :] = u

    uw = jnp.dot(u, W_so_s[:], preferred_element_type=f32)
    ts = Ra[:, D:2 * D] + uw[:, 0:D] + b_sub_ref[:]
    to = Rb[:, D:2 * D] + uw[:, D:2 * D] + b_obj_ref[:]
    atte_s[pl.ds(step * TE, TE), :] = (
        jnp.sum(ts * to, axis=1, keepdims=True) * (1.0 / 16.0))

    eplus = (lax.broadcasted_iota(jnp.int32, (1, TE), 1)
             + step * TE + 1)
    e_row = jnp.where(sub_r == t, eplus, 0)
    e_col = jnp.where(obj_r == t, eplus, 0)
    m_row = jnp.where(b_hot, e_row, 0)
    m_col = jnp.where(a_hot, e_col, 0)
    win_row_t = jnp.max(m_row, axis=1, keepdims=True)
    win_col_t = jnp.max(m_col, axis=1, keepdims=True)

    @pl.when(step == 0)
    def _():
        win_row_s[:] = win_row_t
        win_col_s[:] = win_col_t

    @pl.when(step > 0)
    def _():
        win_row_s[:] = jnp.maximum(win_row_s[:], win_row_t)
        win_col_s[:] = jnp.maximum(win_col_s[:], win_col_t)

    @pl.when(step == _NT - 1)
    def _():
        u_all = u_s[:]
        atte = atte_s[:]
        win_row = win_row_s[:]
        win_col = win_col_s[:]

        colE = lax.broadcasted_iota(jnp.int32, (N, E), 1)
        g_row = ((colE == win_row - 1) & (win_row > 0)).astype(f32)
        g_col = ((colE == win_col - 1) & (win_col > 0)).astype(f32)

        rel_row = jnp.dot(g_row, u_all, preferred_element_type=f32)
        rel_col = jnp.dot(g_col, u_all, preferred_element_type=f32)
        atte_row = jnp.dot(g_row, atte, preferred_element_type=f32)
        atte_col = jnp.dot(g_col, atte, preferred_element_type=f32)

        mask_row = (win_row > 0).astype(f32)
        mask_col = (win_col > 0).astype(f32)

        def _masked_softmax(vec, mask):
            mv = vec * mask
            mx = jnp.max(mv, axis=0, keepdims=True)
            ex = jnp.exp(mv - mx) * mask
            return ex / (jnp.sum(ex, axis=0, keepdims=True) + EPS)

        a_s = _masked_softmax(atte_row, mask_row)
        a_o = _masked_softmax(atte_col, mask_col)

        ctx1 = jnp.sum(phrase * (a_s + a_o), axis=0, keepdims=True)
        ctx2 = jnp.sum(rel_row * a_s + rel_col * a_o, axis=0, keepdims=True)
        delta = (jnp.dot(ctx1, W_phr_ref[0:D, :], preferred_element_type=f32)
                 + jnp.dot(ctx2, W_phr_ref[D:2 * D, :],
                           preferred_element_type=f32)
                 + b_phr_ref[:])

        is_t = (lax.broadcasted_iota(jnp.int32, (N, 1), 0) == t).astype(f32)
        out_phrase_ref[:] = phrase + is_t * delta


def kernel(phrase_feat, rel_feat, rel_conn_mat, target_id, W_rel, b_rel,
           W_sub, b_sub, W_obj, b_obj, W_phr, b_phr):
    tid = jnp.reshape(jnp.asarray(target_id, jnp.int32), (1,))
    conn = jnp.reshape(rel_conn_mat.astype(jnp.int32), (2, 1, E))
    b_rel2 = jnp.reshape(b_rel, (1, D))
    b_sub2 = jnp.reshape(b_sub, (1, D))
    b_obj2 = jnp.reshape(b_obj, (1, D))
    b_phr2 = jnp.reshape(b_phr, (1, D))

    u, out_phrase = pl.pallas_call(
        _sg_kernel,
        grid=(_NT,),
        out_shape=(
            jax.ShapeDtypeStruct((E, D), jnp.float32),
            jax.ShapeDtypeStruct((N, D), jnp.float32),
        ),
        in_specs=[
            pl.BlockSpec(memory_space=pltpu.SMEM),
            pl.BlockSpec((1, 1, TE), lambda i: (0, 0, i)),
            pl.BlockSpec((1, 1, TE), lambda i: (1, 0, i)),
            pl.BlockSpec((N, D), lambda i: (0, 0)),
            pl.BlockSpec((TE, D), lambda i: (i, 0)),
            pl.BlockSpec((3 * D, D), lambda i: (0, 0)),
            pl.BlockSpec((1, D), lambda i: (0, 0)),
            pl.BlockSpec((2 * D, D), lambda i: (0, 0)),
            pl.BlockSpec((1, D), lambda i: (0, 0)),
            pl.BlockSpec((2 * D, D), lambda i: (0, 0)),
            pl.BlockSpec((1, D), lambda i: (0, 0)),
            pl.BlockSpec((2 * D, D), lambda i: (0, 0)),
            pl.BlockSpec((1, D), lambda i: (0, 0)),
        ],
        out_specs=(
            pl.BlockSpec((TE, D), lambda i: (i, 0)),
            pl.BlockSpec((N, D), lambda i: (0, 0)),
        ),
        scratch_shapes=[
            pltpu.VMEM((N, 2 * D), jnp.float32),
            pltpu.VMEM((N, 2 * D), jnp.float32),
            pltpu.VMEM((D, 2 * D), jnp.float32),
            pltpu.VMEM((E, D), jnp.float32),
            pltpu.VMEM((E, 1), jnp.float32),
            pltpu.VMEM((N, 1), jnp.int32),
            pltpu.VMEM((N, 1), jnp.int32),
        ],
    )(tid, conn, conn, phrase_feat, rel_feat, W_rel, b_rel2,
      W_sub, b_sub2, W_obj, b_obj2, W_phr, b_phr2)

    return (out_phrase, u)


# speedup vs baseline: 42.6211x; 1.0168x over previous
import jax
import jax.numpy as jnp
from jax import lax
from jax.experimental import pallas as pl
from jax.experimental.pallas import tpu as pltpu

N = 256
D = 256
E = 4096
TE = 2048
EPS = 1e-06
_NT = E // TE


def _sg_kernel(tid_ref, sub_ref, obj_ref, phrase_ref, rel_ref,
               W_rel_ref, b_rel_ref, W_sub_ref, b_sub_ref,
               W_obj_ref, b_obj_ref, W_phr_ref, b_phr_ref,
               u_ref, out_phrase_ref,
               P_a_s, P_b_s, W_so_s,
               u_s, win_row_s, win_col_s):
    f32 = jnp.float32
    step = pl.program_id(0)
    t = tid_ref[0]
    sub_r = sub_ref[0]
    obj_r = obj_ref[0]
    phrase = phrase_ref[:]

    @pl.when(step == 0)
    def _():
        P_a_s[:, 0:D] = jnp.dot(phrase, W_rel_ref[0:D, :],
                                preferred_element_type=f32)
        P_a_s[:, D:2 * D] = jnp.dot(phrase, W_sub_ref[0:D, :],
                                    preferred_element_type=f32)
        P_b_s[:, 0:D] = jnp.dot(phrase, W_rel_ref[D:2 * D, :],
                                preferred_element_type=f32)
        P_b_s[:, D:2 * D] = jnp.dot(phrase, W_obj_ref[0:D, :],
                                    preferred_element_type=f32)
        W_so_s[:, 0:D] = W_sub_ref[D:2 * D, :]
        W_so_s[:, D:2 * D] = W_obj_ref[D:2 * D, :]

    rowN = lax.broadcasted_iota(jnp.int32, (N, TE), 0)
    a_hot = sub_r == rowN
    b_hot = obj_r == rowN
    AT = a_hot.astype(f32)
    BT = b_hot.astype(f32)
    dn0 = (((0,), (0,)), ((), ()))

    Ra = lax.dot_general(AT, P_a_s[:], dn0, preferred_element_type=f32)
    Rb = lax.dot_general(BT, P_b_s[:], dn0, preferred_element_type=f32)
    u = (Ra[:, 0:D] + Rb[:, 0:D]
         + jnp.dot(rel_ref[:], W_rel_ref[2 * D:3 * D, :],
                   preferred_element_type=f32)
         + b_rel_ref[:])
    u_ref[:] = u
    u_s[pl.ds(step * TE, TE), 0:D] = u

    uw = jnp.dot(u, W_so_s[:], preferred_element_type=f32)
    ts = Ra[:, D:2 * D] + uw[:, 0:D] + b_sub_ref[:]
    to = Rb[:, D:2 * D] + uw[:, D:2 * D] + b_obj_ref[:]
    u_s[pl.ds(step * TE, TE), D:D + 1] = (
        jnp.sum(ts * to, axis=1, keepdims=True) * (1.0 / 16.0))

    eplus = (lax.broadcasted_iota(jnp.int32, (1, TE), 1)
             + step * TE + 1)
    e_row = jnp.where(sub_r == t, eplus, 0)
    e_col = jnp.where(obj_r == t, eplus, 0)
    m_row = jnp.where(b_hot, e_row, 0)
    m_col = jnp.where(a_hot, e_col, 0)
    win_row_t = jnp.max(m_row, axis=1, keepdims=True)
    win_col_t = jnp.max(m_col, axis=1, keepdims=True)

    @pl.when(step == 0)
    def _():
        win_row_s[:] = win_row_t
        win_col_s[:] = win_col_t

    @pl.when(step > 0)
    def _():
        win_row_s[:] = jnp.maximum(win_row_s[:], win_row_t)
        win_col_s[:] = jnp.maximum(win_col_s[:], win_col_t)

    @pl.when(step == _NT - 1)
    def _():
        u_all = u_s[:]
        win_row = win_row_s[:]
        win_col = win_col_s[:]

        colE = lax.broadcasted_iota(jnp.int32, (N, E), 1)
        g_row = (colE == win_row - 1).astype(f32)
        g_col = (colE == win_col - 1).astype(f32)

        sel_row = jnp.dot(g_row, u_all, preferred_element_type=f32)
        sel_col = jnp.dot(g_col, u_all, preferred_element_type=f32)
        rel_row = sel_row[:, 0:D]
        rel_col = sel_col[:, 0:D]
        atte_row = sel_row[:, D:D + 1]
        atte_col = sel_col[:, D:D + 1]

        mask_row = (win_row > 0).astype(f32)
        mask_col = (win_col > 0).astype(f32)

        def _masked_softmax(vec, mask):
            mv = vec * mask
            mx = jnp.max(mv, axis=0, keepdims=True)
            ex = jnp.exp(mv - mx) * mask
            return ex / (jnp.sum(ex, axis=0, keepdims=True) + EPS)

        a_s = _masked_softmax(atte_row, mask_row)
        a_o = _masked_softmax(atte_col, mask_col)

        ctx1 = jnp.sum(phrase * (a_s + a_o), axis=0, keepdims=True)
        ctx2 = jnp.sum(rel_row * a_s + rel_col * a_o, axis=0, keepdims=True)
        delta = (jnp.dot(ctx1, W_phr_ref[0:D, :], preferred_element_type=f32)
                 + jnp.dot(ctx2, W_phr_ref[D:2 * D, :],
                           preferred_element_type=f32)
                 + b_phr_ref[:])

        is_t = (lax.broadcasted_iota(jnp.int32, (N, 1), 0) == t).astype(f32)
        out_phrase_ref[:] = phrase + is_t * delta


def kernel(phrase_feat, rel_feat, rel_conn_mat, target_id, W_rel, b_rel,
           W_sub, b_sub, W_obj, b_obj, W_phr, b_phr):
    tid = jnp.reshape(jnp.asarray(target_id, jnp.int32), (1,))
    conn = jnp.reshape(rel_conn_mat.astype(jnp.int32), (2, 1, E))
    b_rel2 = jnp.reshape(b_rel, (1, D))
    b_sub2 = jnp.reshape(b_sub, (1, D))
    b_obj2 = jnp.reshape(b_obj, (1, D))
    b_phr2 = jnp.reshape(b_phr, (1, D))

    u, out_phrase = pl.pallas_call(
        _sg_kernel,
        grid=(_NT,),
        out_shape=(
            jax.ShapeDtypeStruct((E, D), jnp.float32),
            jax.ShapeDtypeStruct((N, D), jnp.float32),
        ),
        in_specs=[
            pl.BlockSpec(memory_space=pltpu.SMEM),
            pl.BlockSpec((1, 1, TE), lambda i: (0, 0, i)),
            pl.BlockSpec((1, 1, TE), lambda i: (1, 0, i)),
            pl.BlockSpec((N, D), lambda i: (0, 0)),
            pl.BlockSpec((TE, D), lambda i: (i, 0)),
            pl.BlockSpec((3 * D, D), lambda i: (0, 0)),
            pl.BlockSpec((1, D), lambda i: (0, 0)),
            pl.BlockSpec((2 * D, D), lambda i: (0, 0)),
            pl.BlockSpec((1, D), lambda i: (0, 0)),
            pl.BlockSpec((2 * D, D), lambda i: (0, 0)),
            pl.BlockSpec((1, D), lambda i: (0, 0)),
            pl.BlockSpec((2 * D, D), lambda i: (0, 0)),
            pl.BlockSpec((1, D), lambda i: (0, 0)),
        ],
        out_specs=(
            pl.BlockSpec((TE, D), lambda i: (i, 0)),
            pl.BlockSpec((N, D), lambda i: (0, 0)),
        ),
        scratch_shapes=[
            pltpu.VMEM((N, 2 * D), jnp.float32),
            pltpu.VMEM((N, 2 * D), jnp.float32),
            pltpu.VMEM((D, 2 * D), jnp.float32),
            pltpu.VMEM((E, D + 128), jnp.float32),
            pltpu.VMEM((N, 1), jnp.int32),
            pltpu.VMEM((N, 1), jnp.int32),
        ],
    )(tid, conn, conn, phrase_feat, rel_feat, W_rel, b_rel2,
      W_sub, b_sub2, W_obj, b_obj2, W_phr, b_phr2)

    return (out_phrase, u)
